# Initial kernel scaffold; baseline (speedup 1.0000x reference)
import jax
import jax.numpy as jnp
from jax.experimental import pallas as pl


def kernel(predicted_labels, predicted_offsets, gt_boxes, anchors):
    raise NotImplementedError("write your pallas kernel here")


# speedup vs baseline: 5.8211x; 5.8211x over previous
import functools

import jax
import jax.numpy as jnp
from jax.experimental import pallas as pl

_ANCHOR_THRESHOLD = 0.5
_NEG_POS_RATIO = 3


def _body(a_ref, g_ref, lab_ref, off_ref, out_ref, *, n, nobj):
    b = pl.program_id(0)
    nb = pl.num_programs(0)

    ax0 = a_ref[0:1, :]
    ay0 = a_ref[1:2, :]
    ax1 = a_ref[2:3, :]
    ay1 = a_ref[3:4, :]
    g = g_ref[0]
    gx0 = g[:, 0:1]
    gy0 = g[:, 1:2]
    gx1 = g[:, 2:3]
    gy1 = g[:, 3:4]

    iw = jnp.clip(jnp.minimum(gx1, ax1) - jnp.maximum(gx0, ax0), 0.0, None)
    ih = jnp.clip(jnp.minimum(gy1, ay1) - jnp.maximum(gy0, ay0), 0.0, None)
    inter = iw * ih
    aarea = (ax1 - ax0) * (ay1 - ay0)
    garea = (gx1 - gx0) * (gy1 - gy0)
    jac = inter / (aarea + garea - inter)

    jidx = jax.lax.broadcasted_iota(jnp.int32, jac.shape, 0)
    aidx = jax.lax.broadcasted_iota(jnp.int32, jac.shape, 1)

    max_iou = jnp.max(jac, axis=0, keepdims=True)
    gt_ids = jnp.min(jnp.where(jac == max_iou, jidx, nobj), axis=0, keepdims=True)
    row_max = jnp.max(jac, axis=1, keepdims=True)
    a_for_obj = jnp.min(jnp.where(jac == row_max, aidx, n), axis=1, keepdims=True)

    force = aidx == a_for_obj
    forced_j = jnp.max(jnp.where(force, jidx, -1), axis=0, keepdims=True)
    is_forced = forced_j >= 0
    fid = jnp.where(is_forced, forced_j, gt_ids)
    fiou = jnp.where(is_forced, 1.0, max_iou)
    pos = fiou > _ANCHOR_THRESHOLD
    posf = pos.astype(jnp.float32)

    onehot = (jidx == fid).astype(jnp.float32)
    gcx = (gx0 + gx1) * 0.5
    gcy = (gy0 + gy1) * 0.5
    gw = gx1 - gx0
    gh = gy1 - gy0
    mcx = jnp.sum(onehot * gcx, axis=0, keepdims=True)
    mcy = jnp.sum(onehot * gcy, axis=0, keepdims=True)
    mw = jnp.sum(onehot * gw, axis=0, keepdims=True)
    mh = jnp.sum(onehot * gh, axis=0, keepdims=True)

    acx = (ax0 + ax1) * 0.5
    acy = (ay0 + ay1) * 0.5
    aw = ax1 - ax0
    ah = ay1 - ay0
    tx = (mcx - acx) / (aw * 0.1)
    ty = (mcy - acy) / (ah * 0.1)
    tw = jnp.log(mw / aw) * 5.0
    th = jnp.log(mh / ah) * 5.0

    off = off_ref[0]
    box_sum = jnp.float32(0.0)
    for r, t in enumerate((tx, ty, tw, th)):
        d = off[r : r + 1, :] - t
        ad = jnp.abs(d)
        l = jnp.where(ad < 1.0, 0.5 * d * d, ad - 0.5)
        box_sum = box_sum + jnp.sum(jnp.where(pos, l, 0.0))

    npos = jnp.sum(posf)
    x = lab_ref[0]
    bce = jnp.maximum(x, 0.0) - x * posf + jnp.log1p(jnp.exp(-jnp.abs(x)))
    cls_pos = jnp.sum(jnp.where(pos, bce, 0.0))

    v = jnp.where(pos, 0.0, bce)
    u = jax.lax.bitcast_convert_type(v, jnp.int32)
    n_pos_i = jnp.sum(pos.astype(jnp.int32))
    k = jnp.minimum(_NEG_POS_RATIO * n_pos_i, n - n_pos_i)

    def bit_step(i, t):
        cand = t | (jnp.int32(1) << (30 - i))
        c = jnp.sum((u >= cand).astype(jnp.int32))
        return jnp.where(c >= k, cand, t)

    big_t = jax.lax.fori_loop(0, 31, bit_step, jnp.int32(0))
    c_gt = jnp.sum((u > big_t).astype(jnp.int32)).astype(jnp.float32)
    sum_gt = jnp.sum(jnp.where(u > big_t, v, 0.0))
    v_t = jax.lax.bitcast_convert_type(big_t, jnp.float32)
    hard = jnp.where(
        k > 0, sum_gt + (k.astype(jnp.float32) - c_gt) * v_t, jnp.float32(0.0)
    )

    part = jnp.concatenate(
        [
            jnp.reshape(box_sum, (1, 1)),
            jnp.reshape(npos, (1, 1)),
            jnp.reshape(cls_pos, (1, 1)),
            jnp.reshape(hard, (1, 1)),
        ],
        axis=1,
    )

    @pl.when(b == 0)
    def _():
        out_ref[...] = jnp.zeros_like(out_ref)

    out_ref[...] += part

    @pl.when(b == nb - 1)
    def _():
        acc = out_ref[...]
        np_tot = acc[0, 1]
        box_loss = acc[0, 0] / (4.0 * np_tot)
        cls_loss = (acc[0, 2] + acc[0, 3]) / np_tot
        loss = box_loss + cls_loss
        out_ref[...] = jnp.concatenate(
            [
                jnp.reshape(loss, (1, 1)),
                jnp.reshape(box_loss, (1, 1)),
                jnp.reshape(cls_loss, (1, 1)),
                jnp.zeros((1, 1), jnp.float32),
            ],
            axis=1,
        )


def kernel(predicted_labels, predicted_offsets, gt_boxes, anchors):
    b, n = predicted_labels.shape[0], predicted_labels.shape[1]
    nobj = gt_boxes.shape[1]
    a_t = anchors.T
    off_t = jnp.transpose(predicted_offsets, (0, 2, 1))
    lab_t = jnp.transpose(predicted_labels, (0, 2, 1))

    out = pl.pallas_call(
        functools.partial(_body, n=n, nobj=nobj),
        grid=(b,),
        in_specs=[
            pl.BlockSpec((4, n), lambda i: (0, 0)),
            pl.BlockSpec((1, nobj, 4), lambda i: (i, 0, 0)),
            pl.BlockSpec((1, 1, n), lambda i: (i, 0, 0)),
            pl.BlockSpec((1, 4, n), lambda i: (i, 0, 0)),
        ],
        out_specs=pl.BlockSpec((1, 4), lambda i: (0, 0)),
        out_shape=jax.ShapeDtypeStruct((1, 4), jnp.float32),
    )(a_t, gt_boxes, lab_t, off_t)
    return (out[0, 0], out[0, 1], out[0, 2])


# speedup vs baseline: 8.7089x; 1.4961x over previous
import functools

import jax
import jax.numpy as jnp
from jax.experimental import pallas as pl
from jax.experimental.pallas import tpu as pltpu

_ANCHOR_THRESHOLD = 0.5
_NEG_POS_RATIO = 3


def _body(a_ref, g_ref, lab_ref, off_ref, out_ref, v8_ref, *, n, nobj):
    b = pl.program_id(0)
    nb = pl.num_programs(0)

    ax0 = a_ref[0:1, :]
    ay0 = a_ref[1:2, :]
    ax1 = a_ref[2:3, :]
    ay1 = a_ref[3:4, :]
    g = g_ref[0]
    gx0 = g[:, 0:1]
    gy0 = g[:, 1:2]
    gx1 = g[:, 2:3]
    gy1 = g[:, 3:4]

    iw = jnp.clip(jnp.minimum(gx1, ax1) - jnp.maximum(gx0, ax0), 0.0, None)
    ih = jnp.clip(jnp.minimum(gy1, ay1) - jnp.maximum(gy0, ay0), 0.0, None)
    inter = iw * ih
    aarea = (ax1 - ax0) * (ay1 - ay0)
    garea = (gx1 - gx0) * (gy1 - gy0)
    jac = inter / (aarea + garea - inter)

    jidx = jax.lax.broadcasted_iota(jnp.int32, jac.shape, 0)
    aidx = jax.lax.broadcasted_iota(jnp.int32, jac.shape, 1)

    max_iou = jnp.max(jac, axis=0, keepdims=True)
    gt_ids = jnp.min(jnp.where(jac == max_iou, jidx, nobj), axis=0, keepdims=True)
    row_max = jnp.max(jac, axis=1, keepdims=True)
    a_for_obj = jnp.min(jnp.where(jac == row_max, aidx, n), axis=1, keepdims=True)

    force = aidx == a_for_obj
    forced_j = jnp.max(jnp.where(force, jidx, -1), axis=0, keepdims=True)
    is_forced = forced_j >= 0
    fid = jnp.where(is_forced, forced_j, gt_ids)
    fiou = jnp.where(is_forced, 1.0, max_iou)
    pos = fiou > _ANCHOR_THRESHOLD
    posf = pos.astype(jnp.float32)

    onehot = (jidx == fid).astype(jnp.float32)
    gcx = (gx0 + gx1) * 0.5
    gcy = (gy0 + gy1) * 0.5
    gw = gx1 - gx0
    gh = gy1 - gy0
    gtab = jnp.concatenate([gcx, gcy, gw, gh], axis=1)
    matched = jax.lax.dot_general(
        gtab,
        onehot,
        dimension_numbers=(((0,), (0,)), ((), ())),
        preferred_element_type=jnp.float32,
        precision=jax.lax.Precision.HIGHEST,
    )
    mcx = matched[0:1, :]
    mcy = matched[1:2, :]
    mw = matched[2:3, :]
    mh = matched[3:4, :]

    acx = (ax0 + ax1) * 0.5
    acy = (ay0 + ay1) * 0.5
    aw = ax1 - ax0
    ah = ay1 - ay0
    tx = (mcx - acx) / (aw * 0.1)
    ty = (mcy - acy) / (ah * 0.1)
    tw = jnp.log(mw / aw) * 5.0
    th = jnp.log(mh / ah) * 5.0

    off = off_ref[0]
    tcat = jnp.concatenate([tx, ty, tw, th], axis=0)
    d = off - tcat
    ad = jnp.abs(d)
    l = jnp.where(ad < 1.0, 0.5 * d * d, ad - 0.5)
    box_sum = jnp.sum(jnp.where(pos, l, 0.0))

    npos = jnp.sum(posf)
    x = lab_ref[0]
    bce = jnp.maximum(x, 0.0) - x * posf + jnp.log1p(jnp.exp(-jnp.abs(x)))
    cls_pos = jnp.sum(jnp.where(pos, bce, 0.0))

    vrow = jnp.where(pos, 0.0, bce)
    w = n // 8
    for r in range(8):
        v8_ref[r : r + 1, :] = vrow[:, r * w : (r + 1) * w]
    v = v8_ref[...]
    u = jax.lax.bitcast_convert_type(v, jnp.int32)
    n_pos_i = jnp.sum(pos.astype(jnp.int32))
    k = jnp.minimum(_NEG_POS_RATIO * n_pos_i, n - n_pos_i)

    def bit_step(i, t):
        cand = t | (jnp.int32(1) << (30 - i))
        c = jnp.sum((u >= cand).astype(jnp.int32))
        return jnp.where(c >= k, cand, t)

    big_t = jax.lax.fori_loop(0, 31, bit_step, jnp.int32(0))
    c_gt = jnp.sum((u > big_t).astype(jnp.int32)).astype(jnp.float32)
    sum_gt = jnp.sum(jnp.where(u > big_t, v, 0.0))
    v_t = jax.lax.bitcast_convert_type(big_t, jnp.float32)
    hard = jnp.where(
        k > 0, sum_gt + (k.astype(jnp.float32) - c_gt) * v_t, jnp.float32(0.0)
    )

    part = jnp.concatenate(
        [
            jnp.reshape(box_sum, (1, 1)),
            jnp.reshape(npos, (1, 1)),
            jnp.reshape(cls_pos, (1, 1)),
            jnp.reshape(hard, (1, 1)),
        ],
        axis=1,
    )

    @pl.when(b == 0)
    def _():
        out_ref[...] = jnp.zeros_like(out_ref)

    out_ref[...] += part

    @pl.when(b == nb - 1)
    def _():
        acc = out_ref[...]
        np_tot = acc[0, 1]
        box_loss = acc[0, 0] / (4.0 * np_tot)
        cls_loss = (acc[0, 2] + acc[0, 3]) / np_tot
        loss = box_loss + cls_loss
        out_ref[...] = jnp.concatenate(
            [
                jnp.reshape(loss, (1, 1)),
                jnp.reshape(box_loss, (1, 1)),
                jnp.reshape(cls_loss, (1, 1)),
                jnp.zeros((1, 1), jnp.float32),
            ],
            axis=1,
        )


def kernel(predicted_labels, predicted_offsets, gt_boxes, anchors):
    b, n = predicted_labels.shape[0], predicted_labels.shape[1]
    nobj = gt_boxes.shape[1]
    a_t = anchors.T
    off_t = jnp.transpose(predicted_offsets, (0, 2, 1))
    lab_t = jnp.transpose(predicted_labels, (0, 2, 1))

    out = pl.pallas_call(
        functools.partial(_body, n=n, nobj=nobj),
        grid=(b,),
        in_specs=[
            pl.BlockSpec((4, n), lambda i: (0, 0)),
            pl.BlockSpec((1, nobj, 4), lambda i: (i, 0, 0)),
            pl.BlockSpec((1, 1, n), lambda i: (i, 0, 0)),
            pl.BlockSpec((1, 4, n), lambda i: (i, 0, 0)),
        ],
        out_specs=pl.BlockSpec((1, 4), lambda i: (0, 0)),
        out_shape=jax.ShapeDtypeStruct((1, 4), jnp.float32),
        scratch_shapes=[pltpu.VMEM((8, n // 8), jnp.float32)],
    )(a_t, gt_boxes, lab_t, off_t)
    return (out[0, 0], out[0, 1], out[0, 2])


# speedup vs baseline: 9.0509x; 1.0393x over previous
import functools

import jax
import jax.numpy as jnp
from jax.experimental import pallas as pl
from jax.experimental.pallas import tpu as pltpu

_ANCHOR_THRESHOLD = 0.5
_NEG_POS_RATIO = 3


def _body(a_ref, g_ref, lab_ref, off_ref, out_ref, v8_ref, *, n, nobj):
    b = pl.program_id(0)
    nb = pl.num_programs(0)

    ax0 = a_ref[0:1, :]
    ay0 = a_ref[1:2, :]
    ax1 = a_ref[2:3, :]
    ay1 = a_ref[3:4, :]

    parts = []
    for img in range(g_ref.shape[0]):
        parts.append(
            _one_image(a_ref, g_ref, lab_ref, off_ref, v8_ref, img,
                       ax0, ay0, ax1, ay1, n=n, nobj=nobj)
        )
    part = parts[0]
    for p in parts[1:]:
        part = part + p

    @pl.when(b == 0)
    def _():
        out_ref[...] = jnp.zeros_like(out_ref)

    out_ref[...] += part

    @pl.when(b == nb - 1)
    def _():
        acc = out_ref[...]
        np_tot = acc[0, 1]
        box_loss = acc[0, 0] / (4.0 * np_tot)
        cls_loss = (acc[0, 2] + acc[0, 3]) / np_tot
        loss = box_loss + cls_loss
        out_ref[...] = jnp.concatenate(
            [
                jnp.reshape(loss, (1, 1)),
                jnp.reshape(box_loss, (1, 1)),
                jnp.reshape(cls_loss, (1, 1)),
                jnp.zeros((1, 1), jnp.float32),
            ],
            axis=1,
        )


def _one_image(a_ref, g_ref, lab_ref, off_ref, v8_ref, img,
               ax0, ay0, ax1, ay1, *, n, nobj):
    g = g_ref[img]
    gx0 = g[:, 0:1]
    gy0 = g[:, 1:2]
    gx1 = g[:, 2:3]
    gy1 = g[:, 3:4]

    iw = jnp.clip(jnp.minimum(gx1, ax1) - jnp.maximum(gx0, ax0), 0.0, None)
    ih = jnp.clip(jnp.minimum(gy1, ay1) - jnp.maximum(gy0, ay0), 0.0, None)
    inter = iw * ih
    aarea = (ax1 - ax0) * (ay1 - ay0)
    garea = (gx1 - gx0) * (gy1 - gy0)
    jac = inter / (aarea + garea - inter)

    jidx = jax.lax.broadcasted_iota(jnp.int32, jac.shape, 0)
    aidx = jax.lax.broadcasted_iota(jnp.int32, jac.shape, 1)

    max_iou = jnp.max(jac, axis=0, keepdims=True)
    gt_ids = jnp.min(jnp.where(jac == max_iou, jidx, nobj), axis=0, keepdims=True)
    row_max = jnp.max(jac, axis=1, keepdims=True)
    a_for_obj = jnp.min(jnp.where(jac == row_max, aidx, n), axis=1, keepdims=True)

    force = aidx == a_for_obj
    forced_j = jnp.max(jnp.where(force, jidx, -1), axis=0, keepdims=True)
    is_forced = forced_j >= 0
    fid = jnp.where(is_forced, forced_j, gt_ids)
    fiou = jnp.where(is_forced, 1.0, max_iou)
    pos = fiou > _ANCHOR_THRESHOLD
    posf = pos.astype(jnp.float32)

    onehot = (jidx == fid).astype(jnp.float32)
    gcx = (gx0 + gx1) * 0.5
    gcy = (gy0 + gy1) * 0.5
    gw = gx1 - gx0
    gh = gy1 - gy0
    gtab = jnp.concatenate([gcx, gcy, gw, gh], axis=1)
    matched = jax.lax.dot_general(
        gtab,
        onehot,
        dimension_numbers=(((0,), (0,)), ((), ())),
        preferred_element_type=jnp.float32,
        precision=jax.lax.Precision.HIGHEST,
    )
    mcx = matched[0:1, :]
    mcy = matched[1:2, :]
    mw = matched[2:3, :]
    mh = matched[3:4, :]

    acx = (ax0 + ax1) * 0.5
    acy = (ay0 + ay1) * 0.5
    aw = ax1 - ax0
    ah = ay1 - ay0
    tx = (mcx - acx) / (aw * 0.1)
    ty = (mcy - acy) / (ah * 0.1)
    tw = jnp.log(mw / aw) * 5.0
    th = jnp.log(mh / ah) * 5.0

    off = off_ref[img]
    tcat = jnp.concatenate([tx, ty, tw, th], axis=0)
    d = off - tcat
    ad = jnp.abs(d)
    l = jnp.where(ad < 1.0, 0.5 * d * d, ad - 0.5)
    box_sum = jnp.sum(jnp.where(pos, l, 0.0))

    npos = jnp.sum(posf)
    x = lab_ref[img]
    bce = jnp.maximum(x, 0.0) - x * posf + jnp.log1p(jnp.exp(-jnp.abs(x)))
    cls_pos = jnp.sum(jnp.where(pos, bce, 0.0))

    vrow = jnp.where(pos, 0.0, bce)
    w = n // 8
    for r in range(8):
        v8_ref[img, r : r + 1, :] = vrow[:, r * w : (r + 1) * w]
    v = v8_ref[img]
    u = jax.lax.bitcast_convert_type(v, jnp.int32)
    n_pos_i = jnp.sum(pos.astype(jnp.int32))
    k = jnp.minimum(_NEG_POS_RATIO * n_pos_i, n - n_pos_i)

    def bit_step(i, t):
        cand = t | (jnp.int32(1) << (30 - i))
        c = jnp.sum((u >= cand).astype(jnp.int32))
        return jnp.where(c >= k, cand, t)

    big_t = jax.lax.fori_loop(0, 31, bit_step, jnp.int32(0))
    c_gt = jnp.sum((u > big_t).astype(jnp.int32)).astype(jnp.float32)
    sum_gt = jnp.sum(jnp.where(u > big_t, v, 0.0))
    v_t = jax.lax.bitcast_convert_type(big_t, jnp.float32)
    hard = jnp.where(
        k > 0, sum_gt + (k.astype(jnp.float32) - c_gt) * v_t, jnp.float32(0.0)
    )

    return jnp.concatenate(
        [
            jnp.reshape(box_sum, (1, 1)),
            jnp.reshape(npos, (1, 1)),
            jnp.reshape(cls_pos, (1, 1)),
            jnp.reshape(hard, (1, 1)),
        ],
        axis=1,
    )


def kernel(predicted_labels, predicted_offsets, gt_boxes, anchors):
    b, n = predicted_labels.shape[0], predicted_labels.shape[1]
    nobj = gt_boxes.shape[1]
    a_t = anchors.T
    off_t = jnp.transpose(predicted_offsets, (0, 2, 1))
    lab_t = jnp.transpose(predicted_labels, (0, 2, 1))

    out = pl.pallas_call(
        functools.partial(_body, n=n, nobj=nobj),
        grid=(b // 2,),
        in_specs=[
            pl.BlockSpec((4, n), lambda i: (0, 0)),
            pl.BlockSpec((2, nobj, 4), lambda i: (i, 0, 0)),
            pl.BlockSpec((2, 1, n), lambda i: (i, 0, 0)),
            pl.BlockSpec((2, 4, n), lambda i: (i, 0, 0)),
        ],
        out_specs=pl.BlockSpec((1, 4), lambda i: (0, 0)),
        out_shape=jax.ShapeDtypeStruct((1, 4), jnp.float32),
        scratch_shapes=[pltpu.VMEM((2, 8, n // 8), jnp.float32)],
    )(a_t, gt_boxes, lab_t, off_t)
    return (out[0, 0], out[0, 1], out[0, 2])
